# Initial kernel scaffold; baseline (speedup 1.0000x reference)
import jax
import jax.numpy as jnp
from jax.experimental import pallas as pl


def kernel(x, edge_index, W1, b1, W2, b2, Wmu, bmu, Wls, bls, W5, b5, W6, b6):
    raise NotImplementedError("write your pallas kernel here")


# speedup vs baseline: 5.4911x; 5.4911x over previous
import functools

import jax
import jax.numpy as jnp
from jax import lax
from jax.experimental import pallas as pl
from jax.experimental.pallas import tpu as pltpu
from jax.experimental.pallas import tpu_sc as plsc

N = 10000
NP = 10240
D = 128
E = 320000
NC = 2
NS = 16
GRP = 128
EPW = 10240
E_PAD = EPW * NC * NS
GROUPS = EPW // GRP
RPS = NP // NS
DPS = NP // NS

_mesh = plsc.VectorSubcoreMesh(core_axis_name="c", subcore_axis_name="s")


@functools.partial(
    pl.kernel,
    mesh=_mesh,
    out_type=jax.ShapeDtypeStruct((NC * NP,), jnp.float32),
    scratch_types=[
        pltpu.VMEM((GROUPS, GRP), jnp.int32),
        pltpu.VMEM((GRP,), jnp.float32),
        pltpu.VMEM((DPS,), jnp.float32),
        pltpu.VMEM_SHARED((NP,), jnp.float32),
    ],
)
def _sc_degree(dst_hbm, out_hbm, dstv, ones, zbuf, acc):
    c = lax.axis_index("c")
    s = lax.axis_index("s")
    for i in range(GRP // 16):
        ones[pl.ds(i * 16, 16)] = jnp.ones((16,), jnp.float32)
    for i in range(DPS // 16):
        zbuf[pl.ds(i * 16, 16)] = jnp.zeros((16,), jnp.float32)
    pltpu.sync_copy(zbuf, acc.at[pl.ds(s * DPS, DPS)])
    row0 = (c * NS + s) * GROUPS
    pltpu.sync_copy(dst_hbm.at[pl.ds(row0, GROUPS)], dstv)
    plsc.subcore_barrier()

    def body(j, carry):
        pltpu.sync_copy(ones, acc.at[dstv.at[j]], add=True)
        return carry

    lax.fori_loop(0, GROUPS, body, 0)
    plsc.subcore_barrier()
    pltpu.sync_copy(acc.at[pl.ds(s * DPS, DPS)],
                    out_hbm.at[pl.ds(c * NP + s * DPS, DPS)])


@functools.partial(
    pl.kernel,
    mesh=_mesh,
    out_type=jax.ShapeDtypeStruct((NC * NP, D), jnp.float32),
    scratch_types=[
        pltpu.VMEM((GROUPS, GRP), jnp.int32),
        pltpu.VMEM((GROUPS, GRP), jnp.int32),
        pltpu.VMEM((GRP, D), jnp.float32),
        pltpu.VMEM_SHARED((NP, D), jnp.float32),
        pltpu.SemaphoreType.DMA,
    ],
)
def _sc_propagate(g_hbm, src_hbm, dst_hbm, out_hbm, srcv, dstv, rows, acc, sem):
    c = lax.axis_index("c")
    s = lax.axis_index("s")
    pltpu.sync_copy(g_hbm.at[pl.ds(s * RPS, RPS)], acc.at[pl.ds(s * RPS, RPS)])
    row0 = (c * NS + s) * GROUPS
    pltpu.sync_copy(src_hbm.at[pl.ds(row0, GROUPS)], srcv)
    pltpu.sync_copy(dst_hbm.at[pl.ds(row0, GROUPS)], dstv)
    plsc.subcore_barrier()

    def body(j, carry):
        pltpu.async_copy(g_hbm.at[srcv.at[j]], rows, sem).wait()
        pltpu.sync_copy(rows, acc.at[dstv.at[j]], add=True)
        return carry

    lax.fori_loop(0, GROUPS, body, 0)
    plsc.subcore_barrier()
    pltpu.sync_copy(acc.at[pl.ds(s * RPS, RPS)],
                    out_hbm.at[pl.ds(c * NP + s * RPS, RPS)])


def _prescale_body(d0_ref, d1_ref, x_ref, g_ref, dinv_ref):
    deg = d0_ref[...] + d1_ref[...] + 1.0
    dinv = lax.rsqrt(deg)
    dinv_ref[...] = dinv
    g_ref[...] = x_ref[...] * dinv


_BR = 640


def _tc_prescale(d0, d1, x):
    return pl.pallas_call(
        _prescale_body,
        grid=(NP // _BR,),
        in_specs=[
            pl.BlockSpec((_BR, 1), lambda i: (i, 0)),
            pl.BlockSpec((_BR, 1), lambda i: (i, 0)),
            pl.BlockSpec((_BR, D), lambda i: (i, 0)),
        ],
        out_specs=[
            pl.BlockSpec((_BR, D), lambda i: (i, 0)),
            pl.BlockSpec((_BR, 1), lambda i: (i, 0)),
        ],
        out_shape=[
            jax.ShapeDtypeStruct((NP, D), jnp.float32),
            jax.ShapeDtypeStruct((NP, 1), jnp.float32),
        ],
    )(d0, d1, x)


def _layer_body(p_ref, g_ref, dinv_ref, w_ref, b_ref, o_ref, *, relu, scale_out):
    dinv = dinv_ref[...]
    pre = (p_ref[0] + p_ref[1] - g_ref[...]) * dinv
    z = jnp.dot(pre, w_ref[...], preferred_element_type=jnp.float32) + b_ref[...]
    if relu:
        z = jnp.maximum(z, 0.0)
    if scale_out:
        z = z * dinv
    o_ref[...] = z


def _tc_layer(p, g, dinv, w, b, relu, scale_out):
    return pl.pallas_call(
        functools.partial(_layer_body, relu=relu, scale_out=scale_out),
        grid=(NP // _BR,),
        in_specs=[
            pl.BlockSpec((2, _BR, D), lambda i: (0, i, 0)),
            pl.BlockSpec((_BR, D), lambda i: (i, 0)),
            pl.BlockSpec((_BR, 1), lambda i: (i, 0)),
            pl.BlockSpec((D, D), lambda i: (0, 0)),
            pl.BlockSpec((1, D), lambda i: (0, 0)),
        ],
        out_specs=pl.BlockSpec((_BR, D), lambda i: (i, 0)),
        out_shape=jax.ShapeDtypeStruct((NP, D), jnp.float32),
    )(p, g, dinv, w, b)


def kernel(x, edge_index, W1, b1, W2, b2, Wmu, bmu, Wls, bls, W5, b5, W6, b6):
    pad = E_PAD - E
    srcp = jnp.concatenate(
        [edge_index[0], jnp.zeros((pad,), jnp.int32)]).reshape(E_PAD // GRP, GRP)
    dstp = jnp.concatenate(
        [edge_index[1], jnp.full((pad,), N, jnp.int32)]).reshape(E_PAD // GRP, GRP)

    xp = jnp.concatenate([x, jnp.zeros((NP - N, D), jnp.float32)])
    degp = _sc_degree(dstp).reshape(NC, NP)
    d0 = degp[0].reshape(NP, 1)
    d1 = degp[1].reshape(NP, 1)
    g, dinv = _tc_prescale(d0, d1, xp)

    layers = [
        (W1, b1, True, True),
        (W2, b2, True, True),
        (Wmu, bmu, False, True),
        (W5, b5, True, True),
        (W6, b6, True, False),
    ]
    for w, b, relu, scale_out in layers:
        p = _sc_propagate(g, srcp, dstp).reshape(2, NP, D)
        g = _tc_layer(p, g, dinv, w, b.reshape(1, D), relu, scale_out)
    return g[:N]


# speedup vs baseline: 5.9938x; 1.0916x over previous
import functools

import jax
import jax.numpy as jnp
from jax import lax
from jax.experimental import pallas as pl
from jax.experimental.pallas import tpu as pltpu
from jax.experimental.pallas import tpu_sc as plsc

N = 10000
NP = 10240
D = 128
E = 320000
NC = 2
NS = 16
GRP = 128
EPW = 10240
E_PAD = EPW * NC * NS
GROUPS = EPW // GRP
RPS = NP // NS
DPS = NP // NS

_mesh = plsc.VectorSubcoreMesh(core_axis_name="c", subcore_axis_name="s")


@functools.partial(
    pl.kernel,
    mesh=_mesh,
    out_type=jax.ShapeDtypeStruct((NC * NP,), jnp.float32),
    scratch_types=[
        pltpu.VMEM((GROUPS, GRP), jnp.int32),
        pltpu.VMEM((GRP,), jnp.float32),
        pltpu.VMEM((DPS,), jnp.float32),
        pltpu.VMEM_SHARED((NP,), jnp.float32),
    ] + [pltpu.SemaphoreType.DMA] * 8,
)
def _sc_degree(dst_hbm, out_hbm, dstv, ones, zbuf, acc, *sems):
    c = lax.axis_index("c")
    s = lax.axis_index("s")
    for i in range(GRP // 16):
        ones[pl.ds(i * 16, 16)] = jnp.ones((16,), jnp.float32)
    for i in range(DPS // 16):
        zbuf[pl.ds(i * 16, 16)] = jnp.zeros((16,), jnp.float32)
    pltpu.sync_copy(zbuf, acc.at[pl.ds(s * DPS, DPS)])
    row0 = (c * NS + s) * GROUPS
    pltpu.sync_copy(dst_hbm.at[pl.ds(row0, GROUPS)], dstv)
    plsc.subcore_barrier()

    nring = len(sems)

    def body(t, carry):
        for b in range(nring):
            j = t * nring + b

            @pl.when(t > 0)
            def _():
                pltpu.make_async_copy(ones, acc.at[dstv.at[j - nring]],
                                      sems[b]).wait()

            pltpu.async_copy(ones, acc.at[dstv.at[j]], sems[b], add=True)
        return carry

    lax.fori_loop(0, GROUPS // nring, body, 0)
    for b in range(nring):
        j = GROUPS - nring + b
        pltpu.make_async_copy(ones, acc.at[dstv.at[j]], sems[b]).wait()
    plsc.subcore_barrier()
    pltpu.sync_copy(acc.at[pl.ds(s * DPS, DPS)],
                    out_hbm.at[pl.ds(c * NP + s * DPS, DPS)])


@functools.partial(
    pl.kernel,
    mesh=_mesh,
    out_type=jax.ShapeDtypeStruct((NC * NP, D), jnp.float32),
    scratch_types=[
        pltpu.VMEM((GROUPS // 2, GRP), jnp.int32),
        pltpu.VMEM((GROUPS // 2, GRP), jnp.int32),
        pltpu.VMEM((2 * GRP, D), jnp.float32),
        pltpu.VMEM_SHARED((NP, D), jnp.float32),
    ] + [pltpu.SemaphoreType.DMA] * 4,
)
def _sc_propagate(g_hbm, src_hbm, dst_hbm, out_hbm, srcv, dstv, rows, acc,
                  *sems):
    c = lax.axis_index("c")
    s = lax.axis_index("s")
    semg, semsc = sems[:2], sems[2:]
    pltpu.sync_copy(g_hbm.at[pl.ds(s * RPS, RPS)], acc.at[pl.ds(s * RPS, RPS)])
    row0 = (c * NS + s) * GROUPS
    plsc.subcore_barrier()

    buf = [rows.at[pl.ds(b * GRP, GRP)] for b in range(2)]
    half = GROUPS // 2

    for h in range(2):
        pltpu.sync_copy(src_hbm.at[pl.ds(row0 + h * half, half)], srcv)
        pltpu.sync_copy(dst_hbm.at[pl.ds(row0 + h * half, half)], dstv)

        def body(t, carry):
            for b in range(2):
                j = t * 2 + b

                @pl.when(t > 0)
                def _():
                    pltpu.make_async_copy(buf[b], acc.at[dstv.at[j - 2]],
                                          semsc[b]).wait()

                pltpu.async_copy(g_hbm.at[srcv.at[j]], buf[b], semg[b])
            for b in range(2):
                j = t * 2 + b
                pltpu.make_async_copy(g_hbm.at[srcv.at[j]], buf[b],
                                      semg[b]).wait()
                pltpu.async_copy(buf[b], acc.at[dstv.at[j]], semsc[b],
                                 add=True)
            return carry

        lax.fori_loop(0, half // 2, body, 0)
        for b in range(2):
            pltpu.make_async_copy(buf[b], acc.at[dstv.at[half - 2 + b]],
                                  semsc[b]).wait()
    plsc.subcore_barrier()
    pltpu.sync_copy(acc.at[pl.ds(s * RPS, RPS)],
                    out_hbm.at[pl.ds(c * NP + s * RPS, RPS)])


def _prescale_body(d0_ref, d1_ref, x_ref, g_ref, dinv_ref):
    deg = d0_ref[...] + d1_ref[...] + 1.0
    dinv = lax.rsqrt(deg)
    dinv_ref[...] = dinv
    g_ref[...] = x_ref[...] * dinv


_BR = 640


def _tc_prescale(d0, d1, x):
    return pl.pallas_call(
        _prescale_body,
        grid=(NP // _BR,),
        in_specs=[
            pl.BlockSpec((_BR, 1), lambda i: (i, 0)),
            pl.BlockSpec((_BR, 1), lambda i: (i, 0)),
            pl.BlockSpec((_BR, D), lambda i: (i, 0)),
        ],
        out_specs=[
            pl.BlockSpec((_BR, D), lambda i: (i, 0)),
            pl.BlockSpec((_BR, 1), lambda i: (i, 0)),
        ],
        out_shape=[
            jax.ShapeDtypeStruct((NP, D), jnp.float32),
            jax.ShapeDtypeStruct((NP, 1), jnp.float32),
        ],
    )(d0, d1, x)


def _layer_body(p_ref, g_ref, dinv_ref, w_ref, b_ref, o_ref, *, relu, scale_out):
    dinv = dinv_ref[...]
    pre = (p_ref[0] + p_ref[1] - g_ref[...]) * dinv
    z = jnp.dot(pre, w_ref[...], preferred_element_type=jnp.float32) + b_ref[...]
    if relu:
        z = jnp.maximum(z, 0.0)
    if scale_out:
        z = z * dinv
    o_ref[...] = z


def _tc_layer(p, g, dinv, w, b, relu, scale_out):
    return pl.pallas_call(
        functools.partial(_layer_body, relu=relu, scale_out=scale_out),
        grid=(NP // _BR,),
        in_specs=[
            pl.BlockSpec((2, _BR, D), lambda i: (0, i, 0)),
            pl.BlockSpec((_BR, D), lambda i: (i, 0)),
            pl.BlockSpec((_BR, 1), lambda i: (i, 0)),
            pl.BlockSpec((D, D), lambda i: (0, 0)),
            pl.BlockSpec((1, D), lambda i: (0, 0)),
        ],
        out_specs=pl.BlockSpec((_BR, D), lambda i: (i, 0)),
        out_shape=jax.ShapeDtypeStruct((NP, D), jnp.float32),
    )(p, g, dinv, w, b)


def kernel(x, edge_index, W1, b1, W2, b2, Wmu, bmu, Wls, bls, W5, b5, W6, b6):
    pad = E_PAD - E
    srcp = jnp.concatenate(
        [edge_index[0], jnp.zeros((pad,), jnp.int32)]).reshape(E_PAD // GRP, GRP)
    dstp = jnp.concatenate(
        [edge_index[1], jnp.full((pad,), N, jnp.int32)]).reshape(E_PAD // GRP, GRP)

    xp = jnp.concatenate([x, jnp.zeros((NP - N, D), jnp.float32)])
    degp = _sc_degree(dstp).reshape(NC, NP)
    d0 = degp[0].reshape(NP, 1)
    d1 = degp[1].reshape(NP, 1)
    g, dinv = _tc_prescale(d0, d1, xp)

    layers = [
        (W1, b1, True, True),
        (W2, b2, True, True),
        (Wmu, bmu, False, True),
        (W5, b5, True, True),
        (W6, b6, True, False),
    ]
    for w, b, relu, scale_out in layers:
        p = _sc_propagate(g, srcp, dstp).reshape(2, NP, D)
        g = _tc_layer(p, g, dinv, w, b.reshape(1, D), relu, scale_out)
    return g[:N]


# speedup vs baseline: 18.5304x; 3.0916x over previous
import functools

import jax
import jax.numpy as jnp
from jax import lax
from jax.experimental import pallas as pl
from jax.experimental.pallas import tpu as pltpu
from jax.experimental.pallas import tpu_sc as plsc

N = 10000
NP = 10240
D = 128
E = 320000
NC = 2
NS = 16
GRP = 128
EPW = 10240
E_PAD = EPW * NC * NS
GROUPS = EPW // GRP
RPS = NP // NS
DPS = NP // NS

_mesh = plsc.VectorSubcoreMesh(core_axis_name="c", subcore_axis_name="s")


@functools.partial(
    pl.kernel,
    mesh=_mesh,
    out_type=jax.ShapeDtypeStruct((NC * NP,), jnp.float32),
    scratch_types=[
        pltpu.VMEM((GROUPS, GRP), jnp.int32),
        pltpu.VMEM((GRP,), jnp.float32),
        pltpu.VMEM((DPS,), jnp.float32),
        pltpu.VMEM_SHARED((NP,), jnp.float32),
    ] + [pltpu.SemaphoreType.DMA] * 8,
)
def _sc_degree(dst_hbm, out_hbm, dstv, ones, zbuf, acc, *sems):
    c = lax.axis_index("c")
    s = lax.axis_index("s")
    for i in range(GRP // 16):
        ones[pl.ds(i * 16, 16)] = jnp.ones((16,), jnp.float32)
    for i in range(DPS // 16):
        zbuf[pl.ds(i * 16, 16)] = jnp.zeros((16,), jnp.float32)
    pltpu.sync_copy(zbuf, acc.at[pl.ds(s * DPS, DPS)])
    row0 = (c * NS + s) * GROUPS
    pltpu.sync_copy(dst_hbm.at[pl.ds(row0, GROUPS)], dstv)
    plsc.subcore_barrier()

    nring = len(sems)

    def body(t, carry):
        for b in range(nring):
            j = t * nring + b

            @pl.when(t > 0)
            def _():
                pltpu.make_async_copy(ones, acc.at[dstv.at[j - nring]],
                                      sems[b]).wait()

            pltpu.async_copy(ones, acc.at[dstv.at[j]], sems[b], add=True)
        return carry

    lax.fori_loop(0, GROUPS // nring, body, 0)
    for b in range(nring):
        j = GROUPS - nring + b
        pltpu.make_async_copy(ones, acc.at[dstv.at[j]], sems[b]).wait()
    plsc.subcore_barrier()
    pltpu.sync_copy(acc.at[pl.ds(s * DPS, DPS)],
                    out_hbm.at[pl.ds(c * NP + s * DPS, DPS)])


@functools.partial(
    pl.kernel,
    mesh=_mesh,
    out_type=jax.ShapeDtypeStruct((NC * NP, D), jnp.float32),
    scratch_types=[
        pltpu.VMEM((GROUPS // 2, GRP), jnp.int32),
        pltpu.VMEM((GROUPS // 2, GRP), jnp.int32),
        pltpu.VMEM((2 * GRP, D), jnp.float32),
        pltpu.VMEM_SHARED((NP, D), jnp.float32),
    ] + [pltpu.SemaphoreType.DMA] * 4,
)
def _sc_propagate(g_hbm, src_hbm, dst_hbm, out_hbm, srcv, dstv, rows, acc,
                  *sems):
    c = lax.axis_index("c")
    s = lax.axis_index("s")
    semg, semsc = sems[:2], sems[2:]
    pltpu.sync_copy(g_hbm.at[pl.ds(s * RPS, RPS)], acc.at[pl.ds(s * RPS, RPS)])
    row0 = (c * NS + s) * GROUPS
    plsc.subcore_barrier()

    buf = [rows.at[pl.ds(b * GRP, GRP)] for b in range(2)]
    half = GROUPS // 2

    for h in range(2):
        pltpu.sync_copy(src_hbm.at[pl.ds(row0 + h * half, half)], srcv)
        pltpu.sync_copy(dst_hbm.at[pl.ds(row0 + h * half, half)], dstv)

        def body(t, carry):
            for b in range(2):
                j = t * 2 + b

                @pl.when(t > 0)
                def _():
                    pltpu.make_async_copy(buf[b], acc.at[dstv.at[j - 2]],
                                          semsc[b]).wait()

                pltpu.async_copy(g_hbm.at[srcv.at[j]], buf[b], semg[b])
            for b in range(2):
                j = t * 2 + b
                pltpu.make_async_copy(g_hbm.at[srcv.at[j]], buf[b],
                                      semg[b]).wait()
                pltpu.async_copy(buf[b], acc.at[dstv.at[j]], semsc[b],
                                 add=True)
            return carry

        lax.fori_loop(0, half // 2, body, 0)
        for b in range(2):
            pltpu.make_async_copy(buf[b], acc.at[dstv.at[half - 2 + b]],
                                  semsc[b]).wait()
    plsc.subcore_barrier()
    pltpu.sync_copy(acc.at[pl.ds(s * RPS, RPS)],
                    out_hbm.at[pl.ds(c * NP + s * RPS, RPS)])


def _prescale_body(d0_ref, d1_ref, x_ref, g_ref, dinv_ref):
    deg = d0_ref[...] + d1_ref[...] + 1.0
    dinv = lax.rsqrt(deg)
    dinv_ref[...] = dinv
    g_ref[...] = x_ref[...] * dinv


_BR = 640


def _tc_prescale(d0, d1, x):
    return pl.pallas_call(
        _prescale_body,
        grid=(NP // _BR,),
        in_specs=[
            pl.BlockSpec((_BR, 1), lambda i: (i, 0)),
            pl.BlockSpec((_BR, 1), lambda i: (i, 0)),
            pl.BlockSpec((_BR, D), lambda i: (i, 0)),
        ],
        out_specs=[
            pl.BlockSpec((_BR, D), lambda i: (i, 0)),
            pl.BlockSpec((_BR, 1), lambda i: (i, 0)),
        ],
        out_shape=[
            jax.ShapeDtypeStruct((NP, D), jnp.float32),
            jax.ShapeDtypeStruct((NP, 1), jnp.float32),
        ],
    )(d0, d1, x)


def _layer_body(p_ref, g_ref, dinv_ref, w_ref, b_ref, o_ref, *, relu, scale_out):
    dinv = dinv_ref[...]
    pre = (p_ref[0] + p_ref[1] - g_ref[...]) * dinv
    z = jnp.dot(pre, w_ref[...], preferred_element_type=jnp.float32) + b_ref[...]
    if relu:
        z = jnp.maximum(z, 0.0)
    if scale_out:
        z = z * dinv
    o_ref[...] = z


def _tc_layer(p, g, dinv, w, b, relu, scale_out):
    return pl.pallas_call(
        functools.partial(_layer_body, relu=relu, scale_out=scale_out),
        grid=(NP // _BR,),
        in_specs=[
            pl.BlockSpec((2, _BR, D), lambda i: (0, i, 0)),
            pl.BlockSpec((_BR, D), lambda i: (i, 0)),
            pl.BlockSpec((_BR, 1), lambda i: (i, 0)),
            pl.BlockSpec((D, D), lambda i: (0, 0)),
            pl.BlockSpec((1, D), lambda i: (0, 0)),
        ],
        out_specs=pl.BlockSpec((_BR, D), lambda i: (i, 0)),
        out_shape=jax.ShapeDtypeStruct((NP, D), jnp.float32),
    )(p, g, dinv, w, b)


def kernel(x, edge_index, W1, b1, W2, b2, Wmu, bmu, Wls, bls, W5, b5, W6, b6):
    pad = E_PAD - E
    pad_src = jnp.arange(pad, dtype=jnp.int32) % N
    pad_dst = N + (jnp.arange(pad, dtype=jnp.int32) % (NP - N))
    srcp = jnp.concatenate(
        [edge_index[0], pad_src]).reshape(E_PAD // GRP, GRP)
    dstp = jnp.concatenate(
        [edge_index[1], pad_dst]).reshape(E_PAD // GRP, GRP)

    xp = jnp.concatenate([x, jnp.zeros((NP - N, D), jnp.float32)])
    degp = _sc_degree(dstp).reshape(NC, NP)
    d0 = degp[0].reshape(NP, 1)
    d1 = degp[1].reshape(NP, 1)
    g, dinv = _tc_prescale(d0, d1, xp)

    layers = [
        (W1, b1, True, True),
        (W2, b2, True, True),
        (Wmu, bmu, False, True),
        (W5, b5, True, True),
        (W6, b6, True, False),
    ]
    for w, b, relu, scale_out in layers:
        p = _sc_propagate(g, srcp, dstp).reshape(2, NP, D)
        g = _tc_layer(p, g, dinv, w, b.reshape(1, D), relu, scale_out)
    return g[:N]


# speedup vs baseline: 18.6828x; 1.0082x over previous
import functools

import jax
import jax.numpy as jnp
from jax import lax
from jax.experimental import pallas as pl
from jax.experimental.pallas import tpu as pltpu
from jax.experimental.pallas import tpu_sc as plsc

N = 10000
NP = 10240
D = 128
E = 320000
NC = 2
NS = 16
GRP = 128
EPW = 10240
E_PAD = EPW * NC * NS
GROUPS = EPW // GRP
RPS = NP // NS
DPS = NP // NS

_mesh = plsc.VectorSubcoreMesh(core_axis_name="c", subcore_axis_name="s")


@functools.partial(
    pl.kernel,
    mesh=_mesh,
    out_type=jax.ShapeDtypeStruct((NC * NP,), jnp.float32),
    scratch_types=[
        pltpu.VMEM((GROUPS, GRP), jnp.int32),
        pltpu.VMEM((GRP,), jnp.float32),
        pltpu.VMEM((DPS,), jnp.float32),
        pltpu.VMEM_SHARED((NP,), jnp.float32),
    ] + [pltpu.SemaphoreType.DMA] * 8,
)
def _sc_degree(dst_hbm, out_hbm, dstv, ones, zbuf, acc, *sems):
    c = lax.axis_index("c")
    s = lax.axis_index("s")
    for i in range(GRP // 16):
        ones[pl.ds(i * 16, 16)] = jnp.ones((16,), jnp.float32)
    for i in range(DPS // 16):
        zbuf[pl.ds(i * 16, 16)] = jnp.zeros((16,), jnp.float32)
    pltpu.sync_copy(zbuf, acc.at[pl.ds(s * DPS, DPS)])
    row0 = (c * NS + s) * GROUPS
    pltpu.sync_copy(dst_hbm.at[pl.ds(row0, GROUPS)], dstv)
    plsc.subcore_barrier()

    nring = len(sems)

    def body(t, carry):
        for b in range(nring):
            j = t * nring + b

            @pl.when(t > 0)
            def _():
                pltpu.make_async_copy(ones, acc.at[dstv.at[j - nring]],
                                      sems[b]).wait()

            pltpu.async_copy(ones, acc.at[dstv.at[j]], sems[b], add=True)
        return carry

    lax.fori_loop(0, GROUPS // nring, body, 0)
    for b in range(nring):
        j = GROUPS - nring + b
        pltpu.make_async_copy(ones, acc.at[dstv.at[j]], sems[b]).wait()
    plsc.subcore_barrier()
    pltpu.sync_copy(acc.at[pl.ds(s * DPS, DPS)],
                    out_hbm.at[pl.ds(c * NP + s * DPS, DPS)])


@functools.partial(
    pl.kernel,
    mesh=_mesh,
    out_type=jax.ShapeDtypeStruct((NC * NP, D), jnp.float32),
    scratch_types=[
        pltpu.VMEM((GROUPS // 2, GRP), jnp.int32),
        pltpu.VMEM((GROUPS // 2, GRP), jnp.int32),
        pltpu.VMEM((2 * GRP, D), jnp.float32),
        pltpu.VMEM_SHARED((NP, D), jnp.float32),
    ] + [pltpu.SemaphoreType.DMA] * 4,
)
def _sc_propagate(g_hbm, src_hbm, dst_hbm, out_hbm, srcv, dstv, rows, acc,
                  *sems):
    c = lax.axis_index("c")
    s = lax.axis_index("s")
    semg, semsc = sems[:2], sems[2:]
    init = pltpu.async_copy(g_hbm.at[pl.ds(s * RPS, RPS)],
                            acc.at[pl.ds(s * RPS, RPS)], semg[0])
    row0 = (c * NS + s) * GROUPS

    buf = [rows.at[pl.ds(b * GRP, GRP)] for b in range(2)]
    half = GROUPS // 2

    for h in range(2):
        pltpu.sync_copy(src_hbm.at[pl.ds(row0 + h * half, half)], srcv)
        pltpu.sync_copy(dst_hbm.at[pl.ds(row0 + h * half, half)], dstv)
        if h == 0:
            init.wait()
            plsc.subcore_barrier()

        def body(t, carry):
            for b in range(2):
                j = t * 2 + b

                @pl.when(t > 0)
                def _():
                    pltpu.make_async_copy(buf[b], acc.at[dstv.at[j - 2]],
                                          semsc[b]).wait()

                pltpu.async_copy(g_hbm.at[srcv.at[j]], buf[b], semg[b])
            for b in range(2):
                j = t * 2 + b
                pltpu.make_async_copy(g_hbm.at[srcv.at[j]], buf[b],
                                      semg[b]).wait()
                pltpu.async_copy(buf[b], acc.at[dstv.at[j]], semsc[b],
                                 add=True)
            return carry

        lax.fori_loop(0, half // 2, body, 0)
        for b in range(2):
            pltpu.make_async_copy(buf[b], acc.at[dstv.at[half - 2 + b]],
                                  semsc[b]).wait()
    plsc.subcore_barrier()
    pltpu.sync_copy(acc.at[pl.ds(s * RPS, RPS)],
                    out_hbm.at[pl.ds(c * NP + s * RPS, RPS)])


def _prescale_body(d0_ref, d1_ref, x_ref, g_ref, dinv_ref):
    deg = d0_ref[...] + d1_ref[...] + 1.0
    dinv = lax.rsqrt(deg)
    dinv_ref[...] = dinv
    g_ref[...] = x_ref[...] * dinv


_BR = 640


def _tc_prescale(d0, d1, x):
    return pl.pallas_call(
        _prescale_body,
        grid=(NP // _BR,),
        in_specs=[
            pl.BlockSpec((_BR, 1), lambda i: (i, 0)),
            pl.BlockSpec((_BR, 1), lambda i: (i, 0)),
            pl.BlockSpec((_BR, D), lambda i: (i, 0)),
        ],
        out_specs=[
            pl.BlockSpec((_BR, D), lambda i: (i, 0)),
            pl.BlockSpec((_BR, 1), lambda i: (i, 0)),
        ],
        out_shape=[
            jax.ShapeDtypeStruct((NP, D), jnp.float32),
            jax.ShapeDtypeStruct((NP, 1), jnp.float32),
        ],
    )(d0, d1, x)


def _layer_body(p_ref, g_ref, dinv_ref, w_ref, b_ref, o_ref, *, relu, scale_out):
    dinv = dinv_ref[...]
    pre = (p_ref[0] + p_ref[1] - g_ref[...]) * dinv
    z = jnp.dot(pre, w_ref[...], preferred_element_type=jnp.float32) + b_ref[...]
    if relu:
        z = jnp.maximum(z, 0.0)
    if scale_out:
        z = z * dinv
    o_ref[...] = z


def _tc_layer(p, g, dinv, w, b, relu, scale_out):
    return pl.pallas_call(
        functools.partial(_layer_body, relu=relu, scale_out=scale_out),
        grid=(NP // _BR,),
        in_specs=[
            pl.BlockSpec((2, _BR, D), lambda i: (0, i, 0)),
            pl.BlockSpec((_BR, D), lambda i: (i, 0)),
            pl.BlockSpec((_BR, 1), lambda i: (i, 0)),
            pl.BlockSpec((D, D), lambda i: (0, 0)),
            pl.BlockSpec((1, D), lambda i: (0, 0)),
        ],
        out_specs=pl.BlockSpec((_BR, D), lambda i: (i, 0)),
        out_shape=jax.ShapeDtypeStruct((NP, D), jnp.float32),
    )(p, g, dinv, w, b)


def kernel(x, edge_index, W1, b1, W2, b2, Wmu, bmu, Wls, bls, W5, b5, W6, b6):
    pad = E_PAD - E
    pad_src = jnp.arange(pad, dtype=jnp.int32) % N
    pad_dst = N + (jnp.arange(pad, dtype=jnp.int32) % (NP - N))
    srcp = jnp.concatenate(
        [edge_index[0], pad_src]).reshape(E_PAD // GRP, GRP)
    dstp = jnp.concatenate(
        [edge_index[1], pad_dst]).reshape(E_PAD // GRP, GRP)

    xp = jnp.concatenate([x, jnp.zeros((NP - N, D), jnp.float32)])
    degp = _sc_degree(dstp).reshape(NC, NP)
    g, dinv = _tc_prescale(degp[0].reshape(NP, 1), degp[1].reshape(NP, 1), xp)

    layers = [
        (W1, b1, True, True),
        (W2, b2, True, True),
        (Wmu, bmu, False, True),
        (W5, b5, True, True),
        (W6, b6, True, False),
    ]
    for w, b, relu, scale_out in layers:
        p = _sc_propagate(g, srcp, dstp).reshape(2, NP, D)
        g = _tc_layer(p, g, dinv, w, b.reshape(1, D), relu, scale_out)
    return g[:N]
